# Initial kernel scaffold; baseline (speedup 1.0000x reference)
import jax
import jax.numpy as jnp
from jax.experimental import pallas as pl


def kernel(h, edge_index, W, a_s, a_d):
    raise NotImplementedError("write your pallas kernel here")


# speedup vs baseline: 9.1269x; 9.1269x over previous
import functools

import jax
import jax.numpy as jnp
from jax import lax
from jax.experimental import pallas as pl
from jax.experimental.pallas import tpu as pltpu
from jax.experimental.pallas import tpu_sc as plsc

NEG_SLOPE = 0.2

_N = 10000
_E = 160000
_D = 256
_HALF = 128

_NS = 16
_EPT = 10240
_CHUNK = 128
_NCHUNK = _EPT // _CHUNK
_ROWS_PT = 640
_NPAD = _NS * _ROWS_PT


def _tc_body(h_ref, wb_ref, wfull_ref, asd_ref, z0_ref, z1_ref, esed_ref,
             gm_ref):
    i = pl.program_id(0)
    c = pl.program_id(1)
    hb = h_ref[...]
    zb = lax.dot_general(hb, wb_ref[...], (((1,), (1,)), ((), ())),
                         preferred_element_type=jnp.float32)

    @pl.when(c == 0)
    def _():
        z0_ref[...] = zb
        wsd = lax.dot_general(asd_ref[...], wfull_ref[...],
                              (((1,), (0,)), ((), ())),
                              preferred_element_type=jnp.float32)
        esed = lax.dot_general(hb, wsd, (((1,), (1,)), ((), ())),
                               preferred_element_type=jnp.float32)
        esed_ref[...] = esed
        bm = jnp.max(esed[:, 0])

        @pl.when(i == 0)
        def _():
            gm_ref[...] = jnp.full((1, 128), bm, jnp.float32)

        @pl.when(i > 0)
        def _():
            gm_ref[...] = jnp.maximum(gm_ref[...], bm)

    @pl.when(c == 1)
    def _():
        z1_ref[...] = zb


def _tc_compute(h, W, asd):
    n, d = h.shape
    br = 1000
    grid = (n // br, 2)
    return pl.pallas_call(
        _tc_body,
        grid=grid,
        in_specs=[
            pl.BlockSpec((br, d), lambda i, c: (i, 0)),
            pl.BlockSpec((_HALF, d), lambda i, c: (c, 0)),
            pl.BlockSpec((d, d), lambda i, c: (0, 0)),
            pl.BlockSpec((2, d), lambda i, c: (0, 0)),
        ],
        out_specs=[
            pl.BlockSpec((br, _HALF), lambda i, c: (i, 0)),
            pl.BlockSpec((br, _HALF), lambda i, c: (i, 0)),
            pl.BlockSpec((br, 2), lambda i, c: (i, 0)),
            pl.BlockSpec((1, 128), lambda i, c: (0, 0)),
        ],
        out_shape=[
            jax.ShapeDtypeStruct((n, _HALF), jnp.float32),
            jax.ShapeDtypeStruct((n, _HALF), jnp.float32),
            jax.ShapeDtypeStruct((n, 2), jnp.float32),
            jax.ShapeDtypeStruct((1, 128), jnp.float32),
        ],
    )(h, W, W, asd)


def _sc_edge(z0, z1, es, ed, gm, srcp, dstp):
    mesh = plsc.VectorSubcoreMesh(core_axis_name="c", subcore_axis_name="s")

    @functools.partial(
        pl.kernel,
        out_type=jax.ShapeDtypeStruct((_NPAD, _D), jnp.float32),
        mesh=mesh,
        compiler_params=pltpu.CompilerParams(needs_layout_passes=False),
        scratch_types=[
            pltpu.VMEM((_NCHUNK, _CHUNK), jnp.int32),
            pltpu.VMEM((_NCHUNK, _CHUNK), jnp.int32),
            pltpu.VMEM((_CHUNK,), jnp.float32),
            pltpu.VMEM((_CHUNK,), jnp.float32),
            pltpu.VMEM((_CHUNK,), jnp.float32),
            pltpu.VMEM((16,), jnp.float32),
            pltpu.VMEM((_CHUNK, _HALF), jnp.float32),
            pltpu.VMEM((_ROWS_PT,), jnp.float32),
            pltpu.VMEM_SHARED((_N,), jnp.float32),
            pltpu.VMEM_SHARED((_N,), jnp.float32),
            pltpu.VMEM_SHARED((_NPAD, _HALF), jnp.float32),
            pltpu.VMEM_SHARED((_NPAD,), jnp.float32),
        ],
    )
    def k(z0_h, z1_h, es_h, ed_h, gm_h, src_h, dst_h, out_h,
          src2d, dst2d, esg, edg, wbuf, gm_t, rows, dn_t,
          es_sh, ed_sh, out_sh, dn_sh):
        c = lax.axis_index("c")
        s = lax.axis_index("s")

        pltpu.sync_copy(src_h.at[pl.ds(s * _NCHUNK, _NCHUNK)], src2d)
        pltpu.sync_copy(dst_h.at[pl.ds(s * _NCHUNK, _NCHUNK)], dst2d)
        pltpu.sync_copy(gm_h.at[0, pl.ds(0, 16)], gm_t)

        @pl.when(s == 0)
        def _():
            pltpu.sync_copy(es_h, es_sh)
            pltpu.sync_copy(ed_h, ed_sh)

        zeros16 = jnp.zeros((16,), jnp.float32)

        def zrow(r, carry):
            for kk in range(_HALF // 16):
                rows[r, pl.ds(kk * 16, 16)] = zeros16
            return carry

        lax.fori_loop(0, _CHUNK, zrow, 0)

        def zdn(r, carry):
            dn_t[pl.ds(r * 16, 16)] = zeros16
            return carry

        lax.fori_loop(0, _ROWS_PT // 16, zdn, 0)

        for b in range(_ROWS_PT // _CHUNK):
            pltpu.sync_copy(rows, out_sh.at[pl.ds(s * _ROWS_PT + b * _CHUNK,
                                                  _CHUNK)])
        pltpu.sync_copy(dn_t, dn_sh.at[pl.ds(s * _ROWS_PT, _ROWS_PT)])

        gmax = gm_t[pl.ds(0, 16)][0]
        lanes = lax.iota(jnp.int32, 16)
        base_id = s * _EPT

        plsc.subcore_barrier()

        def phb(jj, carry):
            pltpu.sync_copy(es_sh.at[src2d.at[jj]], esg)
            pltpu.sync_copy(ed_sh.at[dst2d.at[jj]], edg)

            @pl.when(c == 0)
            def _():
                pltpu.sync_copy(z0_h.at[src2d.at[jj]], rows)

            @pl.when(c == 1)
            def _():
                pltpu.sync_copy(z1_h.at[src2d.at[jj]], rows)

            for g in range(_CHUNK // 16):
                ess = esg[pl.ds(g * 16, 16)]
                edd = edg[pl.ds(g * 16, 16)]
                e = ess + edd
                e = jnp.maximum(e, NEG_SLOPE * e)
                t = gmax + edd
                cd = jnp.maximum(t, NEG_SLOPE * t)
                w = jnp.exp(e - cd)
                gid = base_id + jj * _CHUNK + g * 16 + lanes
                w = jnp.where(gid < _E, w, 0.0)
                wbuf[pl.ds(g * 16, 16)] = w

            def scale(g, carry2):
                w16 = wbuf[pl.ds(g * 16, 16)]
                for i in range(16):
                    r = g * 16 + i
                    wv = w16[i]
                    for kk in range(_HALF // 16):
                        v = rows[r, pl.ds(kk * 16, 16)]
                        rows[r, pl.ds(kk * 16, 16)] = v * wv
                return carry2

            lax.fori_loop(0, _CHUNK // 16, scale, 0)
            pltpu.sync_copy(rows, out_sh.at[dst2d.at[jj]], add=True)
            pltpu.sync_copy(wbuf, dn_sh.at[dst2d.at[jj]], add=True)
            return carry

        lax.fori_loop(0, _NCHUNK, phb, 0)

        plsc.subcore_barrier()

        pltpu.sync_copy(dn_sh.at[pl.ds(s * _ROWS_PT, _ROWS_PT)], dn_t)

        def nb(b, carry):
            r0 = s * _ROWS_PT + b * _CHUNK
            pltpu.sync_copy(out_sh.at[pl.ds(r0, _CHUNK)], rows)

            def nr(g, carry2):
                d16 = dn_t[pl.ds(b * _CHUNK + g * 16, 16)]
                inv16 = jnp.where(d16 > 0.0, 1.0 / d16, 0.0)
                for i in range(16):
                    r = g * 16 + i
                    inv = inv16[i]
                    for kk in range(_HALF // 16):
                        v = rows[r, pl.ds(kk * 16, 16)]
                        rows[r, pl.ds(kk * 16, 16)] = v * inv
                return carry2

            lax.fori_loop(0, _CHUNK // 16, nr, 0)
            pltpu.sync_copy(rows, out_h.at[pl.ds(r0, _CHUNK),
                                           pl.ds(c * _HALF, _HALF)])
            return carry

        lax.fori_loop(0, _ROWS_PT // _CHUNK, nb, 0)

    return k(z0, z1, es, ed, gm, srcp, dstp)


def kernel(h, edge_index, W, a_s, a_d):
    asd = jnp.concatenate([a_s, a_d], axis=0)
    z0, z1, esed, gm = _tc_compute(h, W, asd)
    es = esed[:, 0]
    ed = esed[:, 1]
    ept = _NS * _EPT
    pad = ept - _E
    zpad = jnp.zeros((pad,), jnp.int32)
    srcp = jnp.concatenate([edge_index[0], zpad]).reshape(_NS * _NCHUNK,
                                                          _CHUNK)
    dstp = jnp.concatenate([edge_index[1], zpad]).reshape(_NS * _NCHUNK,
                                                          _CHUNK)
    outp = _sc_edge(z0, z1, es, ed, gm, srcp, dstp)
    return outp[:_N]


# speedup vs baseline: 10.9498x; 1.1997x over previous
import functools

import jax
import jax.numpy as jnp
from jax import lax
from jax.experimental import pallas as pl
from jax.experimental.pallas import tpu as pltpu
from jax.experimental.pallas import tpu_sc as plsc

NEG_SLOPE = 0.2

_N = 10000
_E = 160000
_D = 256
_HALF = 128

_NS = 16
_EPT = 10240
_CHUNK = 128
_NCHUNK = _EPT // _CHUNK
_ROWS_PT = 640
_NPAD = _NS * _ROWS_PT
_NB = 128


def _tc_body(h_ref, wb_ref, wfull_ref, asd_ref, z0_ref, z1_ref, esed_ref,
             gm_ref):
    i = pl.program_id(0)
    c = pl.program_id(1)
    hb = h_ref[...]
    zb = lax.dot_general(hb, wb_ref[...], (((1,), (1,)), ((), ())),
                         preferred_element_type=jnp.float32)

    @pl.when(c == 0)
    def _():
        z0_ref[...] = zb
        wsd = lax.dot_general(asd_ref[...], wfull_ref[...],
                              (((1,), (0,)), ((), ())),
                              preferred_element_type=jnp.float32)
        esed = lax.dot_general(hb, wsd, (((1,), (1,)), ((), ())),
                               preferred_element_type=jnp.float32)
        esed_ref[...] = esed
        bm = jnp.max(esed[:, 0])

        @pl.when(i == 0)
        def _():
            gm_ref[...] = jnp.full((1, 128), bm, jnp.float32)

        @pl.when(i > 0)
        def _():
            gm_ref[...] = jnp.maximum(gm_ref[...], bm)

    @pl.when(c == 1)
    def _():
        z1_ref[...] = zb


def _tc_compute(h, W, asd):
    n, d = h.shape
    br = 1000
    grid = (n // br, 2)
    return pl.pallas_call(
        _tc_body,
        grid=grid,
        in_specs=[
            pl.BlockSpec((br, d), lambda i, c: (i, 0)),
            pl.BlockSpec((_HALF, d), lambda i, c: (c, 0)),
            pl.BlockSpec((d, d), lambda i, c: (0, 0)),
            pl.BlockSpec((2, d), lambda i, c: (0, 0)),
        ],
        out_specs=[
            pl.BlockSpec((br, _HALF), lambda i, c: (i, 0)),
            pl.BlockSpec((br, _HALF), lambda i, c: (i, 0)),
            pl.BlockSpec((br, 2), lambda i, c: (i, 0)),
            pl.BlockSpec((1, 128), lambda i, c: (0, 0)),
        ],
        out_shape=[
            jax.ShapeDtypeStruct((n, _HALF), jnp.float32),
            jax.ShapeDtypeStruct((n, _HALF), jnp.float32),
            jax.ShapeDtypeStruct((n, 2), jnp.float32),
            jax.ShapeDtypeStruct((1, 128), jnp.float32),
        ],
    )(h, W, W, asd)


def _sc_edge(z0, z1, es, ed, gm, srcp, dstp):
    mesh = plsc.VectorSubcoreMesh(core_axis_name="c", subcore_axis_name="s")

    @functools.partial(
        pl.kernel,
        out_type=jax.ShapeDtypeStruct((_NPAD, _D), jnp.float32),
        mesh=mesh,
        compiler_params=pltpu.CompilerParams(needs_layout_passes=False),
        scratch_types=[
            pltpu.VMEM((2, 1, _CHUNK), jnp.int32),
            pltpu.VMEM((2, 1, _CHUNK), jnp.int32),
            pltpu.VMEM((2, _CHUNK), jnp.float32),
            pltpu.VMEM((2, _CHUNK), jnp.float32),
            pltpu.VMEM((2, _CHUNK), jnp.float32),
            pltpu.VMEM((16,), jnp.float32),
            pltpu.VMEM((2, _CHUNK, _HALF), jnp.float32),
            pltpu.VMEM((_ROWS_PT,), jnp.float32),
            pltpu.SemaphoreType.DMA,
            pltpu.SemaphoreType.DMA,
            pltpu.SemaphoreType.DMA,
            pltpu.SemaphoreType.DMA,
            pltpu.SemaphoreType.DMA,
            pltpu.SemaphoreType.DMA,
            pltpu.VMEM_SHARED((_N,), jnp.float32),
            pltpu.VMEM_SHARED((_N,), jnp.float32),
            pltpu.VMEM_SHARED((_NPAD, _HALF), jnp.float32),
            pltpu.VMEM_SHARED((_NPAD,), jnp.float32),
        ],
    )
    def k(z0_h, z1_h, es_h, ed_h, gm_h, src_h, dst_h, out_h,
          sidx, didx, esg, edg, wbuf, gm_t, rows, dn_t,
          semr0, semr1, seme0, seme1, semd0, semd1,
          es_sh, ed_sh, out_sh, dn_sh):
        c = lax.axis_index("c")
        s = lax.axis_index("s")
        semr = (semr0, semr1)
        seme = (seme0, seme1)
        semd = (semd0, semd1)

        pltpu.sync_copy(gm_h.at[0, pl.ds(0, 16)], gm_t)

        @pl.when(s == 0)
        def _():
            pltpu.sync_copy(es_h, es_sh)
            pltpu.sync_copy(ed_h, ed_sh)

        zeros16 = jnp.zeros((16,), jnp.float32)

        def zrow(r, carry):
            for kk in range(_HALF // 16):
                rows[0, r, pl.ds(kk * 16, 16)] = zeros16
            return carry

        lax.fori_loop(0, _NB, zrow, 0)

        def zdn(r, carry):
            dn_t[pl.ds(r * 16, 16)] = zeros16
            return carry

        lax.fori_loop(0, _ROWS_PT // 16, zdn, 0)

        for b in range(_ROWS_PT // _NB):
            pltpu.sync_copy(rows.at[0], out_sh.at[pl.ds(s * _ROWS_PT + b * _NB,
                                                        _NB)])
        pltpu.sync_copy(dn_t, dn_sh.at[pl.ds(s * _ROWS_PT, _ROWS_PT)])

        gmax = gm_t[pl.ds(0, 16)][0]
        lanes = lax.iota(jnp.int32, 16)
        base_id = s * _EPT
        row0 = s * _NCHUNK

        plsc.subcore_barrier()

        def load_idx(k2, j):
            pltpu.sync_copy(src_h.at[pl.ds(row0 + k2, 1)], sidx.at[j])
            pltpu.sync_copy(dst_h.at[pl.ds(row0 + k2, 1)], didx.at[j])

        def issue_gather(j):
            @pl.when(c == 0)
            def _():
                pltpu.async_copy(z0_h.at[sidx.at[j, 0]], rows.at[j], semr[j])

            @pl.when(c == 1)
            def _():
                pltpu.async_copy(z1_h.at[sidx.at[j, 0]], rows.at[j], semr[j])

            pltpu.async_copy(es_sh.at[sidx.at[j, 0]], esg.at[j], seme[j])
            pltpu.async_copy(ed_sh.at[didx.at[j, 0]], edg.at[j], semd[j])

        def wait_gather(j):
            pltpu.make_async_copy(z0_h.at[sidx.at[j, 0]], rows.at[j],
                                  semr[j]).wait()
            pltpu.make_async_copy(es_sh.at[sidx.at[j, 0]], esg.at[j],
                                  seme[j]).wait()
            pltpu.make_async_copy(ed_sh.at[didx.at[j, 0]], edg.at[j],
                                  semd[j]).wait()

        load_idx(0, 0)
        issue_gather(0)

        def blk(t, carry):
            kc0 = t * 2
            for j in range(2):
                kc = kc0 + j
                q = 1 - j

                @pl.when(kc + 1 < _NCHUNK)
                def _(j=j, q=q, kc=kc):
                    load_idx(kc + 1, q)

                wait_gather(j)

                @pl.when(kc + 1 < _NCHUNK)
                def _(j=j, q=q):
                    issue_gather(q)

                for g in range(_CHUNK // 16):
                    ess = esg[j, pl.ds(g * 16, 16)]
                    edd = edg[j, pl.ds(g * 16, 16)]
                    e = ess + edd
                    e = jnp.maximum(e, NEG_SLOPE * e)
                    tt = gmax + edd
                    cd = jnp.maximum(tt, NEG_SLOPE * tt)
                    w = jnp.exp(e - cd)
                    gid = base_id + kc * _CHUNK + g * 16 + lanes
                    w = jnp.where(gid < _E, w, 0.0)
                    wbuf[j, pl.ds(g * 16, 16)] = w

                def scale(g, carry2, j=j):
                    w16 = wbuf[j, pl.ds(g * 16, 16)]
                    for i in range(16):
                        r = g * 16 + i
                        wv = w16[i]
                        for kk in range(_HALF // 16):
                            v = rows[j, r, pl.ds(kk * 16, 16)]
                            rows[j, r, pl.ds(kk * 16, 16)] = v * wv
                    return carry2

                lax.fori_loop(0, _CHUNK // 16, scale, 0)
                pltpu.sync_copy(rows.at[j], out_sh.at[didx.at[j, 0]],
                                add=True)
                pltpu.sync_copy(wbuf.at[j], dn_sh.at[didx.at[j, 0]],
                                add=True)
            return carry

        lax.fori_loop(0, _NCHUNK // 2, blk, 0)

        plsc.subcore_barrier()

        pltpu.sync_copy(dn_sh.at[pl.ds(s * _ROWS_PT, _ROWS_PT)], dn_t)

        def nb(b, carry):
            r0 = s * _ROWS_PT + b * _NB
            pltpu.sync_copy(out_sh.at[pl.ds(r0, _NB)], rows.at[0])

            def nr(g, carry2):
                d16 = dn_t[pl.ds(b * _NB + g * 16, 16)]
                inv16 = jnp.where(d16 > 0.0, 1.0 / d16, 0.0)
                for i in range(16):
                    r = g * 16 + i
                    inv = inv16[i]
                    for kk in range(_HALF // 16):
                        v = rows[0, r, pl.ds(kk * 16, 16)]
                        rows[0, r, pl.ds(kk * 16, 16)] = v * inv
                return carry2

            lax.fori_loop(0, _NB // 16, nr, 0)
            pltpu.sync_copy(rows.at[0], out_h.at[pl.ds(r0, _NB),
                                                 pl.ds(c * _HALF, _HALF)])
            return carry

        lax.fori_loop(0, _ROWS_PT // _NB, nb, 0)

    return k(z0, z1, es, ed, gm, srcp, dstp)


def kernel(h, edge_index, W, a_s, a_d):
    asd = jnp.concatenate([a_s, a_d], axis=0)
    z0, z1, esed, gm = _tc_compute(h, W, asd)
    es = esed[:, 0]
    ed = esed[:, 1]
    ept = _NS * _EPT
    pad = ept - _E
    zpad = jnp.zeros((pad,), jnp.int32)
    srcp = jnp.concatenate([edge_index[0], zpad]).reshape(_NS * _NCHUNK,
                                                          _CHUNK)
    dstp = jnp.concatenate([edge_index[1], zpad]).reshape(_NS * _NCHUNK,
                                                          _CHUNK)
    outp = _sc_edge(z0, z1, es, ed, gm, srcp, dstp)
    return outp[:_N]


# speedup vs baseline: 10.9617x; 1.0011x over previous
import functools

import jax
import jax.numpy as jnp
from jax import lax
from jax.experimental import pallas as pl
from jax.experimental.pallas import tpu as pltpu
from jax.experimental.pallas import tpu_sc as plsc

NEG_SLOPE = 0.2

_N = 10000
_E = 160000
_D = 256
_HALF = 128

_NS = 16
_EPT = 10240
_CHUNK = 64
_NCHUNK = _EPT // _CHUNK
_NBUF = 4
_ROWS_PT = 640
_NPAD = _NS * _ROWS_PT
_NB = 64


def _tc_body(h_ref, wb_ref, wfull_ref, asd_ref, z0_ref, z1_ref, esed_ref,
             gm_ref):
    i = pl.program_id(0)
    c = pl.program_id(1)
    hb = h_ref[...]
    zb = lax.dot_general(hb, wb_ref[...], (((1,), (1,)), ((), ())),
                         preferred_element_type=jnp.float32)

    @pl.when(c == 0)
    def _():
        z0_ref[...] = zb
        wsd = lax.dot_general(asd_ref[...], wfull_ref[...],
                              (((1,), (0,)), ((), ())),
                              preferred_element_type=jnp.float32)
        esed = lax.dot_general(hb, wsd, (((1,), (1,)), ((), ())),
                               preferred_element_type=jnp.float32)
        esed_ref[...] = esed
        bm = jnp.max(esed[:, 0])

        @pl.when(i == 0)
        def _():
            gm_ref[...] = jnp.full((1, 128), bm, jnp.float32)

        @pl.when(i > 0)
        def _():
            gm_ref[...] = jnp.maximum(gm_ref[...], bm)

    @pl.when(c == 1)
    def _():
        z1_ref[...] = zb


def _tc_compute(h, W, asd):
    n, d = h.shape
    br = 1000
    grid = (n // br, 2)
    return pl.pallas_call(
        _tc_body,
        grid=grid,
        in_specs=[
            pl.BlockSpec((br, d), lambda i, c: (i, 0)),
            pl.BlockSpec((_HALF, d), lambda i, c: (c, 0)),
            pl.BlockSpec((d, d), lambda i, c: (0, 0)),
            pl.BlockSpec((2, d), lambda i, c: (0, 0)),
        ],
        out_specs=[
            pl.BlockSpec((br, _HALF), lambda i, c: (i, 0)),
            pl.BlockSpec((br, _HALF), lambda i, c: (i, 0)),
            pl.BlockSpec((br, 2), lambda i, c: (i, 0)),
            pl.BlockSpec((1, 128), lambda i, c: (0, 0)),
        ],
        out_shape=[
            jax.ShapeDtypeStruct((n, _HALF), jnp.float32),
            jax.ShapeDtypeStruct((n, _HALF), jnp.float32),
            jax.ShapeDtypeStruct((n, 2), jnp.float32),
            jax.ShapeDtypeStruct((1, 128), jnp.float32),
        ],
    )(h, W, W, asd)


def _sc_edge(z0, z1, es, ed, gm, srcp, dstp):
    mesh = plsc.VectorSubcoreMesh(core_axis_name="c", subcore_axis_name="s")

    @functools.partial(
        pl.kernel,
        out_type=jax.ShapeDtypeStruct((_NPAD, _D), jnp.float32),
        mesh=mesh,
        compiler_params=pltpu.CompilerParams(needs_layout_passes=False),
        scratch_types=[
            pltpu.VMEM((_NBUF, 1, _CHUNK), jnp.int32),
            pltpu.VMEM((_NBUF, 1, _CHUNK), jnp.int32),
            pltpu.VMEM((_NBUF, _CHUNK), jnp.float32),
            pltpu.VMEM((_NBUF, _CHUNK), jnp.float32),
            pltpu.VMEM((_NBUF, _CHUNK), jnp.float32),
            pltpu.VMEM((16,), jnp.float32),
            pltpu.VMEM((_NBUF, _CHUNK, _HALF), jnp.float32),
            pltpu.VMEM((_ROWS_PT,), jnp.float32),
            pltpu.SemaphoreType.DMA((_NBUF,)),
            pltpu.SemaphoreType.DMA((_NBUF,)),
            pltpu.SemaphoreType.DMA((_NBUF,)),
            pltpu.SemaphoreType.DMA((_NBUF,)),
            pltpu.SemaphoreType.DMA((_NBUF,)),
            pltpu.SemaphoreType.DMA((_NBUF,)),
            pltpu.SemaphoreType.DMA((_NBUF,)),
            pltpu.VMEM_SHARED((_N,), jnp.float32),
            pltpu.VMEM_SHARED((_N,), jnp.float32),
            pltpu.VMEM_SHARED((_NPAD, _HALF), jnp.float32),
            pltpu.VMEM_SHARED((_NPAD,), jnp.float32),
        ],
    )
    def k(z0_h, z1_h, es_h, ed_h, gm_h, src_h, dst_h, out_h,
          sidx, didx, esg, edg, wbuf, gm_t, rows, dn_t,
          sem_si, sem_di, sem_r, sem_e, sem_d, sem_sr, sem_sw,
          es_sh, ed_sh, out_sh, dn_sh):
        c = lax.axis_index("c")
        s = lax.axis_index("s")

        pltpu.sync_copy(gm_h.at[0, pl.ds(0, 16)], gm_t)

        @pl.when(s == 0)
        def _():
            pltpu.sync_copy(es_h, es_sh)
            pltpu.sync_copy(ed_h, ed_sh)

        zeros16 = jnp.zeros((16,), jnp.float32)

        def zrow(r, carry):
            for kk in range(_HALF // 16):
                rows[0, r, pl.ds(kk * 16, 16)] = zeros16
            return carry

        lax.fori_loop(0, _NB, zrow, 0)

        def zdn(r, carry):
            dn_t[pl.ds(r * 16, 16)] = zeros16
            return carry

        lax.fori_loop(0, _ROWS_PT // 16, zdn, 0)

        for b in range(_ROWS_PT // _NB):
            pltpu.sync_copy(rows.at[0], out_sh.at[pl.ds(s * _ROWS_PT + b * _NB,
                                                        _NB)])
        pltpu.sync_copy(dn_t, dn_sh.at[pl.ds(s * _ROWS_PT, _ROWS_PT)])

        gmax = gm_t[pl.ds(0, 16)][0]
        lanes = lax.iota(jnp.int32, 16)
        base_id = s * _EPT
        row0 = s * _NCHUNK

        plsc.subcore_barrier()

        def issue_gidx(k2, j):
            pltpu.async_copy(src_h.at[pl.ds(row0 + k2, 1)], sidx.at[j],
                             sem_si.at[j])
            pltpu.async_copy(dst_h.at[pl.ds(row0 + k2, 1)], didx.at[j],
                             sem_di.at[j])

        def wait_gidx(j):
            pltpu.make_async_copy(src_h.at[pl.ds(0, 1)], sidx.at[j],
                                  sem_si.at[j]).wait()
            pltpu.make_async_copy(dst_h.at[pl.ds(0, 1)], didx.at[j],
                                  sem_di.at[j]).wait()

        def issue_gather(j):
            @pl.when(c == 0)
            def _():
                pltpu.async_copy(z0_h.at[sidx.at[j, 0]], rows.at[j],
                                 sem_r.at[j])

            @pl.when(c == 1)
            def _():
                pltpu.async_copy(z1_h.at[sidx.at[j, 0]], rows.at[j],
                                 sem_r.at[j])

            pltpu.async_copy(es_sh.at[sidx.at[j, 0]], esg.at[j],
                             sem_e.at[j])
            pltpu.async_copy(ed_sh.at[didx.at[j, 0]], edg.at[j],
                             sem_d.at[j])

        def wait_gather(j):
            pltpu.make_async_copy(z0_h.at[sidx.at[j, 0]], rows.at[j],
                                  sem_r.at[j]).wait()
            pltpu.make_async_copy(es_sh.at[sidx.at[j, 0]], esg.at[j],
                                  sem_e.at[j]).wait()
            pltpu.make_async_copy(ed_sh.at[didx.at[j, 0]], edg.at[j],
                                  sem_d.at[j]).wait()

        def issue_scatter(j):
            pltpu.async_copy(rows.at[j], out_sh.at[didx.at[j, 0]],
                             sem_sr.at[j], add=True)
            pltpu.async_copy(wbuf.at[j], dn_sh.at[didx.at[j, 0]],
                             sem_sw.at[j], add=True)

        def wait_scatter(j):
            pltpu.make_async_copy(rows.at[j], out_sh.at[didx.at[j, 0]],
                                  sem_sr.at[j]).wait()
            pltpu.make_async_copy(wbuf.at[j], dn_sh.at[didx.at[j, 0]],
                                  sem_sw.at[j]).wait()

        issue_gidx(0, 0)
        issue_gidx(1, 1)
        wait_gidx(0)
        issue_gather(0)

        def blk(t, carry):
            kc0 = t * _NBUF
            for j in range(_NBUF):
                kc = kc0 + j

                @pl.when(kc >= 2)
                def _(j=j):
                    wait_scatter((j + 2) % _NBUF)

                @pl.when(kc + 2 < _NCHUNK)
                def _(j=j, kc=kc):
                    issue_gidx(kc + 2, (j + 2) % _NBUF)

                @pl.when(kc + 1 < _NCHUNK)
                def _(j=j):
                    wait_gidx((j + 1) % _NBUF)
                    issue_gather((j + 1) % _NBUF)

                wait_gather(j)

                for g in range(_CHUNK // 16):
                    ess = esg[j, pl.ds(g * 16, 16)]
                    edd = edg[j, pl.ds(g * 16, 16)]
                    e = ess + edd
                    e = jnp.maximum(e, NEG_SLOPE * e)
                    tt = gmax + edd
                    cd = jnp.maximum(tt, NEG_SLOPE * tt)
                    w = jnp.exp(e - cd)
                    gid = base_id + kc * _CHUNK + g * 16 + lanes
                    w = jnp.where(gid < _E, w, 0.0)
                    wbuf[j, pl.ds(g * 16, 16)] = w

                def scale(g, carry2, j=j):
                    w16 = wbuf[j, pl.ds(g * 16, 16)]
                    for i in range(16):
                        r = g * 16 + i
                        wv = w16[i]
                        for kk in range(_HALF // 16):
                            v = rows[j, r, pl.ds(kk * 16, 16)]
                            rows[j, r, pl.ds(kk * 16, 16)] = v * wv
                    return carry2

                lax.fori_loop(0, _CHUNK // 16, scale, 0)
                issue_scatter(j)
            return carry

        lax.fori_loop(0, _NCHUNK // _NBUF, blk, 0)
        wait_scatter((_NCHUNK - 2) % _NBUF)
        wait_scatter((_NCHUNK - 1) % _NBUF)

        plsc.subcore_barrier()

        pltpu.sync_copy(dn_sh.at[pl.ds(s * _ROWS_PT, _ROWS_PT)], dn_t)

        def nb(b, carry):
            r0 = s * _ROWS_PT + b * _NB
            pltpu.sync_copy(out_sh.at[pl.ds(r0, _NB)], rows.at[0])

            def nr(g, carry2):
                d16 = dn_t[pl.ds(b * _NB + g * 16, 16)]
                inv16 = jnp.where(d16 > 0.0, 1.0 / d16, 0.0)
                for i in range(16):
                    r = g * 16 + i
                    inv = inv16[i]
                    for kk in range(_HALF // 16):
                        v = rows[0, r, pl.ds(kk * 16, 16)]
                        rows[0, r, pl.ds(kk * 16, 16)] = v * inv
                return carry2

            lax.fori_loop(0, _NB // 16, nr, 0)
            pltpu.sync_copy(rows.at[0], out_h.at[pl.ds(r0, _NB),
                                                 pl.ds(c * _HALF, _HALF)])
            return carry

        lax.fori_loop(0, _ROWS_PT // _NB, nb, 0)

    return k(z0, z1, es, ed, gm, srcp, dstp)


def kernel(h, edge_index, W, a_s, a_d):
    asd = jnp.concatenate([a_s, a_d], axis=0)
    z0, z1, esed, gm = _tc_compute(h, W, asd)
    es = esed[:, 0]
    ed = esed[:, 1]
    ept = _NS * _EPT
    pad = ept - _E
    zpad = jnp.zeros((pad,), jnp.int32)
    srcp = jnp.concatenate([edge_index[0], zpad]).reshape(_NS * _NCHUNK,
                                                          _CHUNK)
    dstp = jnp.concatenate([edge_index[1], zpad]).reshape(_NS * _NCHUNK,
                                                          _CHUNK)
    outp = _sc_edge(z0, z1, es, ed, gm, srcp, dstp)
    return outp[:_N]
